# baseline (speedup 1.0000x reference)
import functools

import jax
import jax.numpy as jnp
from jax import lax
from jax.experimental import pallas as pl
from jax.experimental.pallas import tpu as pltpu
from jax.experimental.pallas import tpu_sc as plsc


def kernel(x, symbol_library):
    batch, seq_len, d_model = x.shape
    num_workers = 32
    rows_per_worker = seq_len // num_workers
    chunk = 32
    n_chunks = rows_per_worker // chunk

    mesh = plsc.VectorSubcoreMesh(core_axis_name="c", subcore_axis_name="s")

    @functools.partial(
        pl.kernel,
        mesh=mesh,
        out_type=jax.ShapeDtypeStruct((batch, seq_len, d_model), x.dtype),
        scratch_types=[
            pltpu.VMEM((chunk, d_model), jnp.float32),
            pltpu.SemaphoreType.DMA,
        ],
    )
    def sc_broadcast(table_hbm, out_hbm, buf, sem):
        wid = lax.axis_index("s") * 2 + lax.axis_index("c")
        base = wid * rows_per_worker
        for c in range(n_chunks):
            r0 = base + c * chunk
            pltpu.async_copy(table_hbm.at[pl.ds(r0, chunk)], buf, sem).wait()
            for b in range(batch):
                pltpu.sync_copy(buf, out_hbm.at[b, pl.ds(r0, chunk)])

    return sc_broadcast(symbol_library)


# speedup vs baseline: 1.0444x; 1.0444x over previous
import functools

import jax
import jax.numpy as jnp
from jax import lax
from jax.experimental import pallas as pl
from jax.experimental.pallas import tpu as pltpu
from jax.experimental.pallas import tpu_sc as plsc


def kernel(x, symbol_library):
    batch, seq_len, d_model = x.shape
    num_workers = 32
    rows_per_worker = seq_len // num_workers
    chunk = 32
    n_chunks = rows_per_worker // chunk
    nbuf = 2

    mesh = plsc.VectorSubcoreMesh(core_axis_name="c", subcore_axis_name="s")

    @functools.partial(
        pl.kernel,
        mesh=mesh,
        out_type=jax.ShapeDtypeStruct((batch, seq_len, d_model), x.dtype),
        scratch_types=[
            pltpu.VMEM((nbuf, chunk, d_model), jnp.float32),
            pltpu.SemaphoreType.DMA,
            pltpu.SemaphoreType.DMA,
        ],
    )
    def sc_broadcast(table_hbm, out_hbm, bufs, rsem, wsem):
        wid = lax.axis_index("s") * 2 + lax.axis_index("c")
        base = wid * rows_per_worker

        def start_read(c):
            return pltpu.async_copy(
                table_hbm.at[pl.ds(base + c * chunk, chunk)],
                bufs.at[c % nbuf], rsem)

        reads = {0: start_read(0)}
        writes = {}
        for c in range(n_chunks):
            reads[c].wait()
            if c + 1 < n_chunks:
                if c + 1 >= nbuf:
                    for w in writes.pop(c + 1 - nbuf):
                        w.wait()
                reads[c + 1] = start_read(c + 1)
            writes[c] = [
                pltpu.async_copy(
                    bufs.at[c % nbuf],
                    out_hbm.at[b, pl.ds(base + c * chunk, chunk)], wsem)
                for b in range(batch)
            ]
        for c in sorted(writes):
            for w in writes[c]:
                w.wait()

    return sc_broadcast(symbol_library)
